# baseline (speedup 1.0000x reference)
import functools

import jax
import jax.numpy as jnp
from jax import lax
from jax.experimental import pallas as pl
from jax.experimental.pallas import tpu as pltpu
from jax.experimental.pallas import tpu_sc as plsc

_B = 4096
_S = 50
_DIM = 65
_ROWS = _B * _S
_NC = 2
_NS = 16
_NW = _NC * _NS
_RPW = _ROWS // _NW
_CHUNK = 128
_NCHUNK = _RPW // _CHUNK
_PDIM = 80


def _make_gather():
    mesh = plsc.VectorSubcoreMesh(core_axis_name="c", subcore_axis_name="s")

    @functools.partial(
        pl.kernel,
        mesh=mesh,
        compiler_params=pltpu.CompilerParams(use_tc_tiling_on_sc=False),
        out_type=jax.ShapeDtypeStruct((_ROWS, _PDIM), jnp.float32),
        scratch_types=[
            pltpu.VMEM((_CHUNK,), jnp.int32),
            pltpu.VMEM((_CHUNK, _PDIM), jnp.float32),
            pltpu.SemaphoreType.DMA,
        ],
    )
    def gather_kernel(table_hbm, idx_hbm, out_hbm, idx_v, rows_v, sem):
        wid = lax.axis_index("s") * _NC + lax.axis_index("c")
        base = wid * _RPW

        def body(j, carry):
            pltpu.sync_copy(idx_hbm.at[pl.ds(wid * _RPW + j * _CHUNK, _CHUNK)], idx_v)
            pltpu.async_copy(table_hbm.at[idx_v], rows_v, sem).wait()
            pltpu.sync_copy(rows_v, out_hbm.at[pl.ds(base + j * _CHUNK, _CHUNK)])
            return carry

        lax.fori_loop(0, _NCHUNK, body, 0)

    return gather_kernel


_gather = _make_gather()


def kernel(idx, embedding):
    idx_flat = idx.reshape(_ROWS).astype(jnp.int32)
    table = jnp.pad(embedding, ((0, 0), (0, _PDIM - _DIM)))
    out = _gather(table, idx_flat)
    return out[:, :_DIM].reshape(_B, _S, _DIM)


# speedup vs baseline: 2.8910x; 2.8910x over previous
import functools

import jax
import jax.numpy as jnp
from jax import lax
from jax.experimental import pallas as pl
from jax.experimental.pallas import tpu as pltpu
from jax.experimental.pallas import tpu_sc as plsc

_B = 4096
_S = 50
_DIM = 65
_ROWS = _B * _S
_NODES = 1000000
_PDIM = 128
_BN = 2048
_GRID = (_NODES + _BN - 1) // _BN
_NPAD = _GRID * _BN
_NC = 2
_NS = 16
_NW = _NC * _NS
_RPW = _ROWS // _NW
_CH = 128
_NCH = _RPW // _CH


def _transpose_body(in_ref, out_ref):
    x = in_ref[...]
    y = jnp.transpose(x, (1, 0))
    out_ref[:, :_DIM] = y
    out_ref[:, _DIM:] = jnp.zeros((_BN, _PDIM - _DIM), jnp.float32)


_transpose = pl.pallas_call(
    _transpose_body,
    grid=(_GRID,),
    in_specs=[pl.BlockSpec((_DIM, _BN), lambda i: (0, i))],
    out_specs=pl.BlockSpec((_BN, _PDIM), lambda i: (i, 0)),
    out_shape=jax.ShapeDtypeStruct((_NPAD, _PDIM), jnp.float32),
)


def _make_gather():
    mesh = plsc.VectorSubcoreMesh(core_axis_name="c", subcore_axis_name="s")

    @functools.partial(
        pl.kernel,
        mesh=mesh,
        compiler_params=pltpu.CompilerParams(use_tc_tiling_on_sc=True),
        out_type=jax.ShapeDtypeStruct((_ROWS, _PDIM), jnp.float32),
        scratch_types=[
            pltpu.VMEM((_CH,), jnp.int32),
            pltpu.VMEM((_CH, _PDIM), jnp.float32),
            pltpu.SemaphoreType.DMA,
        ],
    )
    def gather_kernel(table_hbm, idx_hbm, out_hbm, idx_v, rows_v, sem):
        wid = lax.axis_index("s") * _NC + lax.axis_index("c")
        base = wid * _RPW

        def body(t, carry):
            pos = base + t * _CH
            pltpu.sync_copy(idx_hbm.at[pl.ds(pos, _CH)], idx_v)
            pltpu.async_copy(table_hbm.at[idx_v], rows_v, sem).wait()
            pltpu.sync_copy(rows_v, out_hbm.at[pl.ds(pos, _CH)])
            return carry

        lax.fori_loop(0, _NCH, body, 0)

    return gather_kernel


_gather = _make_gather()


def kernel(idx, embedding):
    idx_flat = idx.reshape(_ROWS).astype(jnp.int32)
    table128 = _transpose(embedding.T)
    out128 = _gather(table128, idx_flat)
    return out128[:, :_DIM].reshape(_B, _S, _DIM)


# speedup vs baseline: 3.8519x; 1.3324x over previous
import functools

import jax
import jax.numpy as jnp
from jax import lax
from jax.experimental import pallas as pl
from jax.experimental.pallas import tpu as pltpu
from jax.experimental.pallas import tpu_sc as plsc

_B = 4096
_S = 50
_DIM = 65
_ROWS = _B * _S
_NODES = 1000000
_PDIM = 128
_BN = 8192
_GRID = (_NODES + _BN - 1) // _BN
_NPAD = _GRID * _BN
_NC = 2
_NS = 16
_NW = _NC * _NS
_RPW = _ROWS // _NW
_CH = 128
_NCH = _RPW // _CH


def _transpose_body(in_ref, out_ref):
    x = in_ref[...]
    y = jnp.transpose(x, (1, 0))
    out_ref[:, :_DIM] = y
    out_ref[:, _DIM:] = jnp.zeros((_BN, _PDIM - _DIM), jnp.float32)


_transpose = pl.pallas_call(
    _transpose_body,
    grid=(_GRID,),
    in_specs=[pl.BlockSpec((_DIM, _BN), lambda i: (0, i))],
    out_specs=pl.BlockSpec((_BN, _PDIM), lambda i: (i, 0)),
    out_shape=jax.ShapeDtypeStruct((_NPAD, _PDIM), jnp.float32),
)


def _make_gather():
    mesh = plsc.VectorSubcoreMesh(core_axis_name="c", subcore_axis_name="s")

    @functools.partial(
        pl.kernel,
        mesh=mesh,
        compiler_params=pltpu.CompilerParams(use_tc_tiling_on_sc=True),
        out_type=jax.ShapeDtypeStruct((_ROWS, _PDIM), jnp.float32),
        scratch_types=[
            pltpu.VMEM((_CH,), jnp.int32),
            pltpu.VMEM((_CH, _PDIM), jnp.float32),
            pltpu.SemaphoreType.DMA,
        ],
    )
    def gather_kernel(table_hbm, idx_hbm, out_hbm, idx_v, rows_v, sem):
        wid = lax.axis_index("s") * _NC + lax.axis_index("c")
        base = wid * _RPW

        def body(t, carry):
            pos = base + t * _CH
            pltpu.sync_copy(idx_hbm.at[pl.ds(pos, _CH)], idx_v)
            pltpu.async_copy(table_hbm.at[idx_v], rows_v, sem).wait()
            pltpu.sync_copy(rows_v, out_hbm.at[pl.ds(pos, _CH)])
            return carry

        lax.fori_loop(0, _NCH, body, 0)

    return gather_kernel


_gather = _make_gather()


def kernel(idx, embedding):
    idx_flat = idx.reshape(_ROWS).astype(jnp.int32)
    table128 = _transpose(embedding.T)
    out128 = _gather(table128, idx_flat)
    return out128[:, :_DIM].reshape(_B, _S, _DIM)


# speedup vs baseline: 3.9368x; 1.0220x over previous
import functools

import jax
import jax.numpy as jnp
from jax import lax
from jax.experimental import pallas as pl
from jax.experimental.pallas import tpu as pltpu
from jax.experimental.pallas import tpu_sc as plsc

_B = 4096
_S = 50
_DIM = 65
_ROWS = _B * _S
_NODES = 1000000
_PDIM = 128
_BN = 16384
_GRID = (_NODES + _BN - 1) // _BN
_NPAD = _GRID * _BN
_NC = 2
_NS = 16
_NW = _NC * _NS
_RPW = _ROWS // _NW
_CH = 128
_NCH = _RPW // _CH


def _transpose_body(in_ref, out_ref):
    x = in_ref[...]
    y = jnp.transpose(x, (1, 0))
    out_ref[:, :_DIM] = y
    out_ref[:, _DIM:] = jnp.zeros((_BN, _PDIM - _DIM), jnp.float32)


_transpose = pl.pallas_call(
    _transpose_body,
    grid=(_GRID,),
    in_specs=[pl.BlockSpec((_DIM, _BN), lambda i: (0, i))],
    out_specs=pl.BlockSpec((_BN, _PDIM), lambda i: (i, 0)),
    out_shape=jax.ShapeDtypeStruct((_NPAD, _PDIM), jnp.float32),
)


def _make_gather():
    mesh = plsc.VectorSubcoreMesh(core_axis_name="c", subcore_axis_name="s")

    @functools.partial(
        pl.kernel,
        mesh=mesh,
        compiler_params=pltpu.CompilerParams(use_tc_tiling_on_sc=True),
        out_type=jax.ShapeDtypeStruct((_ROWS, _PDIM), jnp.float32),
        scratch_types=[
            pltpu.VMEM((_CH,), jnp.int32),
            pltpu.VMEM((_CH, _PDIM), jnp.float32),
            pltpu.SemaphoreType.DMA,
        ],
    )
    def gather_kernel(table_hbm, idx_hbm, out_hbm, idx_v, rows_v, sem):
        wid = lax.axis_index("s") * _NC + lax.axis_index("c")
        base = wid * _RPW

        def body(t, carry):
            pos = base + t * _CH
            pltpu.sync_copy(idx_hbm.at[pl.ds(pos, _CH)], idx_v)
            pltpu.async_copy(table_hbm.at[idx_v], rows_v, sem).wait()
            pltpu.sync_copy(rows_v, out_hbm.at[pl.ds(pos, _CH)])
            return carry

        lax.fori_loop(0, _NCH, body, 0)

    return gather_kernel


_gather = _make_gather()


def kernel(idx, embedding):
    idx_flat = idx.reshape(_ROWS).astype(jnp.int32)
    table128 = _transpose(embedding.T)
    out128 = _gather(table128, idx_flat)
    return out128[:, :_DIM].reshape(_B, _S, _DIM)
